# Initial kernel scaffold; baseline (speedup 1.0000x reference)
import functools

import jax
import jax.numpy as jnp
from jax import lax
from jax.experimental import pallas as pl
from jax.experimental.pallas import tpu as pltpu
from jax.experimental.pallas import tpu_sc as plsc

H = 64
NA = 819200
NSV = 16384
ROWS = 100001
HPAD = 100096
NC = 2
NS = 16
NW = NC * NS
A_PER = NA // NW
SV_PER = NSV // NW
SV_CH = 128
SV_NCH = SV_PER // SV_CH

_MESH = plsc.VectorSubcoreMesh(core_axis_name="c", subcore_axis_name="s")


@functools.partial(
    pl.kernel,
    out_type=jax.ShapeDtypeStruct((NW, HPAD), jnp.float32),
    mesh=_MESH,
    scratch_types=[
        pltpu.VMEM((A_PER,), jnp.int32),
        pltpu.VMEM((HPAD,), jnp.float32),
    ],
)
def _hist(a_hbm, counts_hbm, idx_v, hist_v):
    cid = lax.axis_index("c")
    sid = lax.axis_index("s")
    wid = sid * NC + cid

    zero16 = jnp.zeros((16,), jnp.float32)

    @plsc.parallel_loop(0, HPAD, step=16)
    def _(i):
        hist_v[pl.ds(i, 16)] = zero16

    pltpu.sync_copy(a_hbm.at[pl.ds(wid * A_PER, A_PER)], idx_v)

    def body(i, carry):
        idx16 = idx_v[pl.ds(i * 16, 16)]
        cnt, last = plsc.scan_count(idx16)
        plsc.addupdate_scatter(
            hist_v, [idx16], cnt.astype(jnp.float32), mask=last
        )
        return carry

    lax.fori_loop(0, A_PER // 16, body, 0, unroll=8)

    pltpu.sync_copy(hist_v, counts_hbm.at[wid])


@functools.partial(
    pl.kernel,
    out_type=jax.ShapeDtypeStruct((NSV, H), jnp.float32),
    mesh=_MESH,
    scratch_types=[
        pltpu.VMEM((SV_NCH, SV_CH), jnp.int32),
        pltpu.VMEM((SV_NCH, SV_CH), jnp.int32),
        pltpu.VMEM((SV_PER, H), jnp.float32),
        pltpu.VMEM((SV_PER, H), jnp.float32),
        pltpu.SemaphoreType.DMA,
        pltpu.SemaphoreType.DMA,
    ],
)
def _sv(s_hbm, v_hbm, wsh_hbm, wvh_hbm, out_hbm,
        sidx_v, vidx_v, srow_v, vrow_v, ssem, vsem):
    cid = lax.axis_index("c")
    sid = lax.axis_index("s")
    wid = sid * NC + cid
    base = wid * SV_PER

    pltpu.sync_copy(s_hbm.at[wid], sidx_v)
    pltpu.sync_copy(v_hbm.at[wid], vidx_v)

    copies = []
    for j in range(SV_NCH):
        copies.append(pltpu.async_copy(
            wsh_hbm.at[sidx_v.at[j]], srow_v.at[pl.ds(j * SV_CH, SV_CH)],
            ssem))
        copies.append(pltpu.async_copy(
            wvh_hbm.at[vidx_v.at[j]], vrow_v.at[pl.ds(j * SV_CH, SV_CH)],
            vsem))
    for c in copies:
        c.wait()

    @plsc.parallel_loop(0, SV_PER, step=1, unroll=2)
    def _(r):
        for j in range(H // 16):
            sl = pl.ds(j * 16, 16)
            srow_v[r, sl] = srow_v[r, sl] + vrow_v[r, sl]

    pltpu.sync_copy(srow_v, out_hbm.at[pl.ds(base, SV_PER)])


_BR = HPAD // 32


def _matvec_body(c_ref, w_ref, o_ref):
    g = pl.program_id(0)
    c = c_ref[...]
    w = w_ref[...]
    rows = g * _BR + lax.broadcasted_iota(jnp.int32, (_BR, 1), 0)
    w = jnp.where(rows < ROWS, w, 0.0)
    csum = jnp.sum(c, axis=0).reshape(1, _BR)
    part = jnp.dot(csum, w, preferred_element_type=jnp.float32)

    @pl.when(g == 0)
    def _():
        o_ref[...] = jnp.zeros_like(o_ref)

    o_ref[...] += part


def _matvec(counts, wah):
    out = pl.pallas_call(
        _matvec_body,
        grid=(HPAD // _BR,),
        in_specs=[
            pl.BlockSpec((NW, _BR), lambda g: (0, g)),
            pl.BlockSpec((_BR, H), lambda g: (g, 0)),
        ],
        out_specs=pl.BlockSpec((1, H), lambda g: (0, 0)),
        out_shape=jax.ShapeDtypeStruct((1, H), jnp.float32),
    )(counts, wah)
    return out[0]


def kernel(a, s, v, Wah, Wsh, Wvh):
    a = a.astype(jnp.int32)
    s = s.astype(jnp.int32).reshape(NW, SV_NCH, SV_CH)
    v = v.astype(jnp.int32).reshape(NW, SV_NCH, SV_CH)
    counts = _hist(a)
    a_emb = _matvec(counts, Wah)
    sv_emb = _sv(s, v, Wsh, Wvh)
    return (a_emb, sv_emb)


# speedup vs baseline: 13.9290x; 13.9290x over previous
import functools

import jax
import jax.numpy as jnp
from jax import lax
from jax.experimental import pallas as pl
from jax.experimental.pallas import tpu as pltpu
from jax.experimental.pallas import tpu_sc as plsc

H = 64
NA = 819200
NSV = 16384
ROWS = 100001
HPAD = 100096
NC = 2
NS = 16
NW = NC * NS
A_PER = NA // NW
SV_PER = NSV // NW
SV_CH = 128
SV_NCH = SV_PER // SV_CH

_MESH = plsc.VectorSubcoreMesh(core_axis_name="c", subcore_axis_name="s")


@functools.partial(
    pl.kernel,
    out_type=jax.ShapeDtypeStruct((NW, HPAD), jnp.float32),
    mesh=_MESH,
    scratch_types=[
        pltpu.VMEM((A_PER,), jnp.int32),
        pltpu.VMEM((HPAD,), jnp.float32),
    ],
    compiler_params=pltpu.CompilerParams(needs_layout_passes=False),
)
def _hist(a_hbm, counts_hbm, idx_v, hist_v):
    cid = lax.axis_index("c")
    sid = lax.axis_index("s")
    wid = sid * NC + cid

    zero16 = jnp.zeros((16,), jnp.float32)

    @plsc.parallel_loop(0, HPAD, step=16)
    def _(i):
        hist_v[pl.ds(i, 16)] = zero16

    pltpu.sync_copy(a_hbm.at[pl.ds(wid * A_PER, A_PER)], idx_v)

    def body(i, carry):
        idx16 = idx_v[pl.ds(i * 16, 16)]
        cnt, last = plsc.scan_count(idx16)
        plsc.addupdate_scatter(
            hist_v, [idx16], cnt.astype(jnp.float32), mask=last
        )
        return carry

    lax.fori_loop(0, A_PER // 16, body, 0, unroll=8)

    pltpu.sync_copy(hist_v, counts_hbm.at[wid])


@functools.partial(
    pl.kernel,
    out_type=jax.ShapeDtypeStruct((NSV, H), jnp.float32),
    mesh=_MESH,
    scratch_types=[
        pltpu.VMEM((SV_NCH, SV_CH), jnp.int32),
        pltpu.VMEM((SV_NCH, SV_CH), jnp.int32),
        pltpu.VMEM((SV_PER, H), jnp.float32),
        pltpu.VMEM((SV_PER, H), jnp.float32),
        pltpu.SemaphoreType.DMA,
        pltpu.SemaphoreType.DMA,
    ],
    compiler_params=pltpu.CompilerParams(use_tc_tiling_on_sc=False),
)
def _sv(s_hbm, v_hbm, wsh_hbm, wvh_hbm, out_hbm,
        sidx_v, vidx_v, srow_v, vrow_v, ssem, vsem):
    cid = lax.axis_index("c")
    sid = lax.axis_index("s")
    wid = sid * NC + cid
    base = wid * SV_PER

    pltpu.sync_copy(s_hbm.at[wid], sidx_v)
    pltpu.sync_copy(v_hbm.at[wid], vidx_v)

    copies = []
    for j in range(SV_NCH):
        copies.append(pltpu.async_copy(
            wsh_hbm.at[sidx_v.at[j]], srow_v.at[pl.ds(j * SV_CH, SV_CH)],
            ssem))
        copies.append(pltpu.async_copy(
            wvh_hbm.at[vidx_v.at[j]], vrow_v.at[pl.ds(j * SV_CH, SV_CH)],
            vsem))
    for c in copies:
        c.wait()

    @plsc.parallel_loop(0, SV_PER, step=1, unroll=2)
    def _(r):
        for j in range(H // 16):
            sl = pl.ds(j * 16, 16)
            srow_v[r, sl] = srow_v[r, sl] + vrow_v[r, sl]

    pltpu.sync_copy(srow_v, out_hbm.at[pl.ds(base, SV_PER)])


_BR = 5888


def _matvec_body(c_ref, w_ref, o_ref):
    g = pl.program_id(0)
    c = c_ref[...]
    w = w_ref[...]
    rows = g * _BR + lax.broadcasted_iota(jnp.int32, (_BR, 1), 0)
    w = jnp.where(rows < ROWS, w, 0.0)
    csum = jnp.sum(c, axis=0).reshape(1, _BR)
    part = jnp.dot(csum, w, preferred_element_type=jnp.float32)

    @pl.when(g == 0)
    def _():
        o_ref[...] = jnp.zeros_like(o_ref)

    o_ref[...] += part


def _matvec(counts, wah):
    out = pl.pallas_call(
        _matvec_body,
        grid=(HPAD // _BR,),
        in_specs=[
            pl.BlockSpec((NW, _BR), lambda g: (0, g)),
            pl.BlockSpec((_BR, H), lambda g: (g, 0)),
        ],
        out_specs=pl.BlockSpec((1, H), lambda g: (0, 0)),
        out_shape=jax.ShapeDtypeStruct((1, H), jnp.float32),
    )(counts, wah)
    return out[0]


def kernel(a, s, v, Wah, Wsh, Wvh):
    a = a.astype(jnp.int32)
    s = s.astype(jnp.int32).reshape(NW, SV_NCH, SV_CH)
    v = v.astype(jnp.int32).reshape(NW, SV_NCH, SV_CH)
    counts = _hist(a)
    a_emb = _matvec(counts, Wah)
    sv_emb = _sv(s, v, Wsh, Wvh)
    return (a_emb, sv_emb)


# speedup vs baseline: 17.2752x; 1.2402x over previous
import functools

import jax
import jax.numpy as jnp
from jax import lax
from jax.experimental import pallas as pl
from jax.experimental.pallas import tpu as pltpu
from jax.experimental.pallas import tpu_sc as plsc

H = 64
NA = 819200
NSV = 16384
ROWS = 100001
HPAD = 100096
NC = 2
NS = 16
NW = NC * NS
A_PER = NA // NW
SV_PER = NSV // NW
SV_CH = 128
SV_NCH = SV_PER // SV_CH

_MESH = plsc.VectorSubcoreMesh(core_axis_name="c", subcore_axis_name="s")


@functools.partial(
    pl.kernel,
    out_type=jax.ShapeDtypeStruct((NW, HPAD), jnp.int32),
    mesh=_MESH,
    scratch_types=[
        pltpu.VMEM((A_PER,), jnp.int32),
        pltpu.VMEM((HPAD,), jnp.int32),
    ],
    compiler_params=pltpu.CompilerParams(needs_layout_passes=False),
)
def _hist(a_hbm, counts_hbm, idx_v, hist_v):
    cid = lax.axis_index("c")
    sid = lax.axis_index("s")
    wid = sid * NC + cid

    zero16 = jnp.zeros((16,), jnp.int32)

    @plsc.parallel_loop(0, HPAD, step=16, unroll=16)
    def _(i):
        hist_v[pl.ds(i, 16)] = zero16

    pltpu.sync_copy(a_hbm.at[pl.ds(wid * A_PER, A_PER)], idx_v)

    def body(i, carry):
        idx16 = idx_v[pl.ds(i * 16, 16)]
        cnt, last = plsc.scan_count(idx16)
        plsc.addupdate_scatter(hist_v, [idx16], cnt, mask=last)
        return carry

    lax.fori_loop(0, A_PER // 16, body, 0, unroll=8)

    pltpu.sync_copy(hist_v, counts_hbm.at[wid])


@functools.partial(
    pl.kernel,
    out_type=jax.ShapeDtypeStruct((NSV, H), jnp.float32),
    mesh=_MESH,
    scratch_types=[
        pltpu.VMEM((SV_NCH, SV_CH), jnp.int32),
        pltpu.VMEM((SV_NCH, SV_CH), jnp.int32),
        pltpu.VMEM((SV_PER, H), jnp.float32),
        pltpu.VMEM((SV_PER, H), jnp.float32),
        pltpu.SemaphoreType.DMA,
        pltpu.SemaphoreType.DMA,
    ],
    compiler_params=pltpu.CompilerParams(use_tc_tiling_on_sc=False),
)
def _sv(s_hbm, v_hbm, wsh_hbm, wvh_hbm, out_hbm,
        sidx_v, vidx_v, srow_v, vrow_v, ssem, vsem):
    cid = lax.axis_index("c")
    sid = lax.axis_index("s")
    wid = sid * NC + cid
    base = wid * SV_PER

    pltpu.sync_copy(s_hbm.at[wid], sidx_v)
    pltpu.sync_copy(v_hbm.at[wid], vidx_v)

    copies = []
    for j in range(SV_NCH):
        copies.append(pltpu.async_copy(
            wsh_hbm.at[sidx_v.at[j]], srow_v.at[pl.ds(j * SV_CH, SV_CH)],
            ssem))
        copies.append(pltpu.async_copy(
            wvh_hbm.at[vidx_v.at[j]], vrow_v.at[pl.ds(j * SV_CH, SV_CH)],
            vsem))
    for c in copies:
        c.wait()

    @plsc.parallel_loop(0, SV_PER, step=1, unroll=2)
    def _(r):
        for j in range(H // 16):
            sl = pl.ds(j * 16, 16)
            srow_v[r, sl] = srow_v[r, sl] + vrow_v[r, sl]

    pltpu.sync_copy(srow_v, out_hbm.at[pl.ds(base, SV_PER)])


_BR = 5888


def _matvec_body(c_ref, wt_ref, o_ref):
    g = pl.program_id(0)
    c = c_ref[...]
    wt = wt_ref[...]
    cols = g * _BR + lax.broadcasted_iota(jnp.int32, (1, _BR), 1)
    wt = jnp.where(cols < ROWS, wt, 0.0)
    csum = jnp.sum(c, axis=0).reshape(1, _BR).astype(jnp.float32)
    part = lax.dot_general(
        csum, wt, (((1,), (1,)), ((), ())),
        preferred_element_type=jnp.float32)

    @pl.when(g == 0)
    def _():
        o_ref[...] = jnp.zeros_like(o_ref)

    o_ref[...] += part


def _matvec(counts, wah_t):
    out = pl.pallas_call(
        _matvec_body,
        grid=(HPAD // _BR,),
        in_specs=[
            pl.BlockSpec((NW, _BR), lambda g: (0, g)),
            pl.BlockSpec((H, _BR), lambda g: (0, g)),
        ],
        out_specs=pl.BlockSpec((1, H), lambda g: (0, 0)),
        out_shape=jax.ShapeDtypeStruct((1, H), jnp.float32),
    )(counts, wah_t)
    return out[0]


def kernel(a, s, v, Wah, Wsh, Wvh):
    a = a.astype(jnp.int32)
    s = s.astype(jnp.int32).reshape(NW, SV_NCH, SV_CH)
    v = v.astype(jnp.int32).reshape(NW, SV_NCH, SV_CH)
    counts = _hist(a)
    a_emb = _matvec(counts, Wah.T)
    sv_emb = _sv(s, v, Wsh, Wvh)
    return (a_emb, sv_emb)


# speedup vs baseline: 17.3677x; 1.0054x over previous
import functools

import jax
import jax.numpy as jnp
from jax import lax
from jax.experimental import pallas as pl
from jax.experimental.pallas import tpu as pltpu
from jax.experimental.pallas import tpu_sc as plsc

H = 64
NA = 819200
NSV = 16384
ROWS = 100001
HPAD = 100096
NC = 2
NS = 16
NW = NC * NS
A_PER = NA // NW
SV_PER = NSV // NW
SV_CH = 128
SV_NCH = SV_PER // SV_CH

_MESH = plsc.VectorSubcoreMesh(core_axis_name="c", subcore_axis_name="s")


@functools.partial(
    pl.kernel,
    out_type=jax.ShapeDtypeStruct((NW, HPAD), jnp.int32),
    mesh=_MESH,
    scratch_types=[
        pltpu.VMEM((A_PER,), jnp.int32),
        pltpu.VMEM((HPAD,), jnp.int32),
    ],
    compiler_params=pltpu.CompilerParams(needs_layout_passes=False),
)
def _hist(a_hbm, counts_hbm, idx_v, hist_v):
    cid = lax.axis_index("c")
    sid = lax.axis_index("s")
    wid = sid * NC + cid

    zero16 = jnp.zeros((16,), jnp.int32)

    @plsc.parallel_loop(0, HPAD, step=16, unroll=16)
    def _(i):
        hist_v[pl.ds(i, 16)] = zero16

    pltpu.sync_copy(a_hbm.at[pl.ds(wid * A_PER, A_PER)], idx_v)

    ones = jnp.ones((16,), jnp.int32)

    def body(i, carry):
        idx16 = idx_v[pl.ds(i * 16, 16)]
        plsc.addupdate_scatter(hist_v, [idx16], ones)
        return carry

    lax.fori_loop(0, A_PER // 16, body, 0, unroll=8)

    pltpu.sync_copy(hist_v, counts_hbm.at[wid])


@functools.partial(
    pl.kernel,
    out_type=jax.ShapeDtypeStruct((NSV, H), jnp.float32),
    mesh=_MESH,
    scratch_types=[
        pltpu.VMEM((SV_NCH, SV_CH), jnp.int32),
        pltpu.VMEM((SV_NCH, SV_CH), jnp.int32),
        pltpu.VMEM((SV_PER, H), jnp.float32),
        pltpu.VMEM((SV_PER, H), jnp.float32),
        pltpu.SemaphoreType.DMA,
        pltpu.SemaphoreType.DMA,
    ],
    compiler_params=pltpu.CompilerParams(use_tc_tiling_on_sc=False),
)
def _sv(s_hbm, v_hbm, wsh_hbm, wvh_hbm, out_hbm,
        sidx_v, vidx_v, srow_v, vrow_v, ssem, vsem):
    cid = lax.axis_index("c")
    sid = lax.axis_index("s")
    wid = sid * NC + cid
    base = wid * SV_PER

    pltpu.sync_copy(s_hbm.at[wid], sidx_v)
    pltpu.sync_copy(v_hbm.at[wid], vidx_v)

    copies = []
    for j in range(SV_NCH):
        copies.append(pltpu.async_copy(
            wsh_hbm.at[sidx_v.at[j]], srow_v.at[pl.ds(j * SV_CH, SV_CH)],
            ssem))
        copies.append(pltpu.async_copy(
            wvh_hbm.at[vidx_v.at[j]], vrow_v.at[pl.ds(j * SV_CH, SV_CH)],
            vsem))
    for c in copies:
        c.wait()

    @plsc.parallel_loop(0, SV_PER, step=1, unroll=2)
    def _(r):
        for j in range(H // 16):
            sl = pl.ds(j * 16, 16)
            srow_v[r, sl] = srow_v[r, sl] + vrow_v[r, sl]

    pltpu.sync_copy(srow_v, out_hbm.at[pl.ds(base, SV_PER)])


_BR = 5888


def _matvec_body(c_ref, wt_ref, o_ref):
    g = pl.program_id(0)
    c = c_ref[...]
    wt = wt_ref[...]
    cols = g * _BR + lax.broadcasted_iota(jnp.int32, (1, _BR), 1)
    wt = jnp.where(cols < ROWS, wt, 0.0)
    csum = jnp.sum(c, axis=0).reshape(1, _BR).astype(jnp.float32)
    part = lax.dot_general(
        csum, wt, (((1,), (1,)), ((), ())),
        preferred_element_type=jnp.float32)

    @pl.when(g == 0)
    def _():
        o_ref[...] = jnp.zeros_like(o_ref)

    o_ref[...] += part


def _matvec(counts, wah_t):
    out = pl.pallas_call(
        _matvec_body,
        grid=(HPAD // _BR,),
        in_specs=[
            pl.BlockSpec((NW, _BR), lambda g: (0, g)),
            pl.BlockSpec((H, _BR), lambda g: (0, g)),
        ],
        out_specs=pl.BlockSpec((1, H), lambda g: (0, 0)),
        out_shape=jax.ShapeDtypeStruct((1, H), jnp.float32),
    )(counts, wah_t)
    return out[0]


def kernel(a, s, v, Wah, Wsh, Wvh):
    a = a.astype(jnp.int32)
    s = s.astype(jnp.int32).reshape(NW, SV_NCH, SV_CH)
    v = v.astype(jnp.int32).reshape(NW, SV_NCH, SV_CH)
    counts = _hist(a)
    a_emb = _matvec(counts, Wah.T)
    sv_emb = _sv(s, v, Wsh, Wvh)
    return (a_emb, sv_emb)


# speedup vs baseline: 17.8298x; 1.0266x over previous
import functools

import jax
import jax.numpy as jnp
from jax import lax
from jax.experimental import pallas as pl
from jax.experimental.pallas import tpu as pltpu
from jax.experimental.pallas import tpu_sc as plsc

H = 64
NA = 819200
NSV = 16384
ROWS = 100001
HR = 896
HPAD = HR * 128
HCH = 128
HSL = HR // 16
NC = 2
NS = 16
NW = NC * NS
A_PER = NA // NW
A_NCH = 8
A_CH = A_PER // A_NCH
SV_PER = NSV // NW
SV_CH = 128
SV_NCH = SV_PER // SV_CH

_MESH = plsc.VectorSubcoreMesh(core_axis_name="c", subcore_axis_name="s")


@functools.partial(
    pl.kernel,
    out_type=jax.ShapeDtypeStruct((NC, HR, 128), jnp.int32),
    mesh=_MESH,
    scratch_types=[
        pltpu.VMEM((2, A_CH), jnp.int32),
        pltpu.VMEM((HR, 128), jnp.int32),
        pltpu.VMEM((HR // HCH, HCH), jnp.int32),
        pltpu.VMEM_SHARED((HR, 128), jnp.int32),
        pltpu.SemaphoreType.DMA,
        pltpu.SemaphoreType.DMA,
    ],
    compiler_params=pltpu.CompilerParams(needs_layout_passes=False),
)
def _hist(a_hbm, counts_hbm, idx_v, hist_v, iota_v, sh_hist, sem0, sem1):
    cid = lax.axis_index("c")
    sid = lax.axis_index("s")
    wid = sid * NC + cid
    base = wid * A_PER
    sems = (sem0, sem1)

    dmas = [pltpu.async_copy(a_hbm.at[pl.ds(base, A_CH)], idx_v.at[0],
                             sem0)]

    zero16 = jnp.zeros((16,), jnp.int32)

    @plsc.parallel_loop(0, HR, step=1, unroll=4)
    def _(r):
        for j in range(128 // 16):
            hist_v[r, pl.ds(j * 16, 16)] = zero16

    iota16 = lax.iota(jnp.int32, 16)
    for k in range(HR // HCH):
        for j in range(HCH // 16):
            iota_v[k, pl.ds(j * 16, 16)] = iota16 + (k * HCH + j * 16)

    pltpu.sync_copy(hist_v.at[pl.ds(sid * HSL, HSL)],
                    sh_hist.at[pl.ds(sid * HSL, HSL)])

    ones = jnp.ones((16,), jnp.int32)

    for c in range(A_NCH):
        if c + 1 < A_NCH:
            dmas.append(pltpu.async_copy(
                a_hbm.at[pl.ds(base + (c + 1) * A_CH, A_CH)],
                idx_v.at[(c + 1) % 2], sems[(c + 1) % 2]))
        dmas[c].wait()

        @plsc.parallel_loop(0, A_CH // 16, step=1, unroll=8)
        def _(i):
            idx16 = idx_v[c % 2, pl.ds(i * 16, 16)]
            plsc.addupdate_scatter(hist_v, [idx16 // 128, idx16 % 128],
                                   ones)

    plsc.subcore_barrier()

    for k in range(HR // HCH):
        pltpu.sync_copy(hist_v.at[pl.ds(k * HCH, HCH)],
                        sh_hist.at[iota_v.at[k]], add=True)

    plsc.subcore_barrier()

    pltpu.sync_copy(sh_hist.at[pl.ds(sid * HSL, HSL)],
                    hist_v.at[pl.ds(0, HSL)])
    pltpu.sync_copy(hist_v.at[pl.ds(0, HSL)],
                    counts_hbm.at[cid].at[pl.ds(sid * HSL, HSL)])


@functools.partial(
    pl.kernel,
    out_type=jax.ShapeDtypeStruct((NSV, H), jnp.float32),
    mesh=_MESH,
    scratch_types=[
        pltpu.VMEM((SV_NCH, SV_CH), jnp.int32),
        pltpu.VMEM((SV_NCH, SV_CH), jnp.int32),
        pltpu.VMEM((SV_PER, H), jnp.float32),
        pltpu.VMEM((SV_PER, H), jnp.float32),
        pltpu.SemaphoreType.DMA,
        pltpu.SemaphoreType.DMA,
    ],
    compiler_params=pltpu.CompilerParams(use_tc_tiling_on_sc=False),
)
def _sv(s_hbm, v_hbm, wsh_hbm, wvh_hbm, out_hbm,
        sidx_v, vidx_v, srow_v, vrow_v, ssem, vsem):
    cid = lax.axis_index("c")
    sid = lax.axis_index("s")
    wid = sid * NC + cid
    base = wid * SV_PER

    pltpu.sync_copy(s_hbm.at[wid], sidx_v)
    pltpu.sync_copy(v_hbm.at[wid], vidx_v)

    copies = []
    for j in range(SV_NCH):
        copies.append(pltpu.async_copy(
            wsh_hbm.at[sidx_v.at[j]], srow_v.at[pl.ds(j * SV_CH, SV_CH)],
            ssem))
        copies.append(pltpu.async_copy(
            wvh_hbm.at[vidx_v.at[j]], vrow_v.at[pl.ds(j * SV_CH, SV_CH)],
            vsem))
    for c in copies:
        c.wait()

    @plsc.parallel_loop(0, SV_PER, step=1, unroll=2)
    def _(r):
        for j in range(H // 16):
            sl = pl.ds(j * 16, 16)
            srow_v[r, sl] = srow_v[r, sl] + vrow_v[r, sl]

    pltpu.sync_copy(srow_v, out_hbm.at[pl.ds(base, SV_PER)])


_BR = 7168
_MV_STEPS = 14


def _matvec_body(c_ref, wt_ref, o_ref):
    g = pl.program_id(0)
    c3 = c_ref[...]
    wt = wt_ref[...]
    cols = g * _BR + lax.broadcasted_iota(jnp.int32, (1, _BR), 1)
    wt = jnp.where(cols < ROWS, wt, 0.0)
    csum = (c3[0] + c3[1]).astype(jnp.float32)
    part = jnp.zeros((1, H), jnp.float32)
    for r in range(HSL):
        part += lax.dot_general(
            csum[r:r + 1, :], wt[:, r * 128:(r + 1) * 128],
            (((1,), (1,)), ((), ())),
            preferred_element_type=jnp.float32)

    @pl.when(g == 0)
    def _():
        o_ref[...] = jnp.zeros_like(o_ref)

    o_ref[...] += part


def _matvec(counts3, wah_t):
    out = pl.pallas_call(
        _matvec_body,
        grid=(_MV_STEPS,),
        in_specs=[
            pl.BlockSpec((NC, HSL, 128), lambda g: (0, g, 0)),
            pl.BlockSpec((H, _BR), lambda g: (0, g)),
        ],
        out_specs=pl.BlockSpec((1, H), lambda g: (0, 0)),
        out_shape=jax.ShapeDtypeStruct((1, H), jnp.float32),
    )(counts3, wah_t)
    return out[0]


def kernel(a, s, v, Wah, Wsh, Wvh):
    a = a.astype(jnp.int32)
    s = s.astype(jnp.int32).reshape(NW, SV_NCH, SV_CH)
    v = v.astype(jnp.int32).reshape(NW, SV_NCH, SV_CH)
    counts3 = _hist(a)
    a_emb = _matvec(counts3, Wah.T)
    sv_emb = _sv(s, v, Wsh, Wvh)
    return (a_emb, sv_emb)
